# baseline (speedup 1.0000x reference)
import functools

import jax
import jax.numpy as jnp
from jax import lax
from jax.experimental import pallas as pl
from jax.experimental.pallas import tpu as pltpu
from jax.experimental.pallas import tpu_sc as plsc

VOCAB = 1000000
EMB_DIM = 32
BATCH = 4096
SEQ = 50
TOTAL = BATCH * SEQ

_info = plsc.get_sparse_core_info()
NC, NS = _info.num_cores, _info.num_subcores
NW = NC * NS
ROWS_PER_W = BATCH // NW
ROWS_PER_CHUNK = 32
CHUNK = ROWS_PER_CHUNK * SEQ
N_CHUNKS = ROWS_PER_W // ROWS_PER_CHUNK


def _emb_kernel(idx_hbm, table_hbm, out_hbm,
                idx0, idx1, rows0, rows1, gsem, wsem):
    wid = lax.axis_index("s") * NC + lax.axis_index("c")
    row_base = wid * ROWS_PER_W
    idx_v = [idx0, idx1]
    rows_v = [rows0, rows1]

    def stage_idx(g, b):
        off = (row_base + g * ROWS_PER_CHUNK) * SEQ
        pltpu.sync_copy(idx_hbm.at[pl.ds(off, CHUNK)], idx_v[b])

    def fire_gather(g, b):
        return pltpu.async_copy(table_hbm.at[idx_v[b]], rows_v[b], gsem)

    def fire_writes(g, b):
        r0 = row_base + g * ROWS_PER_CHUNK
        return [
            pltpu.async_copy(
                rows_v[b].at[pl.ds(r * SEQ, SEQ)],
                out_hbm.at[r0 + r, :, :],
                wsem,
            )
            for r in range(ROWS_PER_CHUNK)
        ]

    stage_idx(0, 0)
    gathers = [fire_gather(0, 0)]
    writes = []
    for g in range(N_CHUNKS):
        if g + 1 < N_CHUNKS:
            b = (g + 1) % 2
            stage_idx(g + 1, b)
            if g >= 1:
                for d in writes[g - 1]:
                    d.wait()
            gathers.append(fire_gather(g + 1, b))
        gathers[g].wait()
        writes.append(fire_writes(g, g % 2))
    for d in writes[N_CHUNKS - 2]:
        d.wait()
    for d in writes[N_CHUNKS - 1]:
        d.wait()


@jax.jit
def _emb(word_ids, table):
    guarded = jnp.where(word_ids < 0, word_ids + VOCAB, word_ids)
    idx = jnp.clip(guarded, 0, VOCAB - 1).reshape(TOTAL)
    mesh = plsc.VectorSubcoreMesh(core_axis_name="c", subcore_axis_name="s")
    k = functools.partial(
        pl.kernel,
        mesh=mesh,
        out_type=jax.ShapeDtypeStruct((BATCH, SEQ, EMB_DIM), jnp.float32),
        scratch_types=[
            pltpu.VMEM((CHUNK,), jnp.int32),
            pltpu.VMEM((CHUNK,), jnp.int32),
            pltpu.VMEM((CHUNK, EMB_DIM), jnp.float32),
            pltpu.VMEM((CHUNK, EMB_DIM), jnp.float32),
            pltpu.SemaphoreType.DMA,
            pltpu.SemaphoreType.DMA,
        ],
        compiler_params=pltpu.CompilerParams(use_tc_tiling_on_sc=False),
    )(_emb_kernel)
    return k(idx, table)


def kernel(word_ids, table):
    return _emb(word_ids, table)
